# Initial kernel scaffold; baseline (speedup 1.0000x reference)
import jax
import jax.numpy as jnp
from jax.experimental import pallas as pl


def kernel(p_t, p_i, e_t, e_i, m_t, m_i, attn_W1, attn_b1, attn_W2, attn_b2, gate_W1, gate_b1, gate_W2, gate_b2):
    raise NotImplementedError("write your pallas kernel here")


# speedup vs baseline: 1.2116x; 1.2116x over previous
import functools

import jax
import jax.numpy as jnp
from jax.experimental import pallas as pl
from jax.experimental.pallas import tpu as pltpu

B = 16384
D = 64
AGR_T = 0.3
SEM_T = 0.3
IL_COEF = 0.7
BL_COEF = 0.1
RZ_COEF = 0.01

BLOCK = 2048


def _softplus(x):
    return jnp.maximum(x, 0.0) + jnp.log1p(jnp.exp(-jnp.abs(x)))


def _silu(x):
    return x * jax.nn.sigmoid(x)


def _fused_kernel(et_ref, ei_ref, mt_ref, mi_ref,
                  aW1_ref, ab1_ref, aW2_ref, ab2_ref,
                  gW1_ref, gb1_ref, gW2_ref, gb2_ref,
                  mask_ref, loss_ref, acc_ref):
    i = pl.program_id(0)
    nblk = pl.num_programs(0)

    @pl.when(i == 0)
    def _init():
        acc_ref[0] = 0.0
        acc_ref[1] = 0.0
        acc_ref[2] = 0.0

    et = et_ref[...]
    ei = ei_ref[...]
    mt = mt_ref[...]
    mi = mi_ref[...]

    aW1 = aW1_ref[...]
    ab1 = ab1_ref[...]
    aw2 = aW2_ref[...]
    ab2 = ab2_ref[0, 0]
    gW1 = gW1_ref[...]
    gb1 = gb1_ref[...]
    gw2_0 = gW2_ref[0:1, :]
    gw2_1 = gW2_ref[1:2, :]
    gb2_0 = gb2_ref[0, 0]
    gb2_1 = gb2_ref[0, 1]

    dot_ti = jnp.sum(mt * mi, axis=1)
    nt = jnp.sum(mt * mt, axis=1)
    ni = jnp.sum(mi * mi, axis=1)
    clip = dot_ti * jax.lax.rsqrt(nt) * jax.lax.rsqrt(ni)
    sem1 = clip > SEM_T

    def score(x):
        h = _silu(jnp.dot(x, aW1, preferred_element_type=jnp.float32) + ab1)
        return jnp.sum(h * aw2, axis=1) + ab2

    s_et = score(et)
    s_ei = score(ei)
    s_mt = score(mt)
    s_mi = score(mi)

    gate_in = (s_et[:, None] * et + s_ei[:, None] * ei
               + s_mt[:, None] * mt + s_mi[:, None] * mi)
    g = _silu(jnp.dot(gate_in, gW1, preferred_element_type=jnp.float32) + gb1)
    l0 = jnp.sum(g * gw2_0, axis=1) + gb2_0
    l1 = jnp.sum(g * gw2_1, axis=1) + gb2_1

    d = l0 - l1
    logp0 = -_softplus(-d)
    logp1 = -_softplus(d)
    p0 = jnp.exp(logp0)
    p1 = jnp.exp(logp1)
    lse = jnp.maximum(l0, l1) + jnp.log1p(jnp.exp(-jnp.abs(d)))

    picked = jnp.where(sem1, logp1, logp0)
    acc_ref[0] += jnp.sum(picked)
    acc_ref[1] += jnp.sum(lse * lse)
    acc_ref[2] += jnp.sum((l1 > l0).astype(jnp.float32))

    col = jax.lax.broadcasted_iota(jnp.int32, (p0.shape[0], 4), 1)
    mask_ref[...] = jnp.where(col < 2, p0[:, None], p1[:, None])

    @pl.when(i == nblk - 1)
    def _final():
        inv_b = 1.0 / B
        interaction = IL_COEF * (-(acc_ref[0] * inv_b))
        router_z = RZ_COEF * (RZ_COEF * (acc_ref[1] * inv_b))
        d1 = acc_ref[2] * inv_b
        balance = BL_COEF * (d1 - 0.5) * (d1 - 0.5)
        loss_ref[0, 0] = interaction + router_z + balance


@jax.jit
def _run(e_t, e_i, m_t, m_i, attn_W1, attn_b1, attn_W2, attn_b2,
         gate_W1, gate_b1, gate_W2, gate_b2):
    nblk = B // BLOCK
    row_spec = pl.BlockSpec((BLOCK, D), lambda i: (i, 0))
    full = lambda shape: pl.BlockSpec(shape, lambda i: (0,) * len(shape))

    mask, loss = pl.pallas_call(
        _fused_kernel,
        grid=(nblk,),
        in_specs=[
            row_spec, row_spec, row_spec, row_spec,
            full((D, D)), full((1, D)), full((1, D)), full((1, 1)),
            full((D, D)), full((1, D)), full((2, D)), full((1, 2)),
        ],
        out_specs=[
            pl.BlockSpec((BLOCK, 4), lambda i: (i, 0)),
            pl.BlockSpec(memory_space=pltpu.SMEM),
        ],
        out_shape=[
            jax.ShapeDtypeStruct((B, 4), jnp.float32),
            jax.ShapeDtypeStruct((1, 1), jnp.float32),
        ],
        scratch_shapes=[pltpu.SMEM((3,), jnp.float32)],
    )(e_t, e_i, m_t, m_i,
      attn_W1, attn_b1.reshape(1, D), attn_W2.reshape(1, D),
      attn_b2.reshape(1, 1),
      gate_W1, gate_b1.reshape(1, D), gate_W2.T, gate_b2.reshape(1, 2))
    return mask, loss[0, 0]


def kernel(p_t, p_i, e_t, e_i, m_t, m_i, attn_W1, attn_b1, attn_W2, attn_b2,
           gate_W1, gate_b1, gate_W2, gate_b2):
    return _run(e_t, e_i, m_t, m_i, attn_W1, attn_b1, attn_W2, attn_b2,
                gate_W1, gate_b1, gate_W2, gate_b2)
